# Initial kernel scaffold; baseline (speedup 1.0000x reference)
import jax
import jax.numpy as jnp
from jax.experimental import pallas as pl


def kernel(tokenized_sentence, table):
    raise NotImplementedError("write your pallas kernel here")


# speedup vs baseline: 4.2464x; 4.2464x over previous
import functools

import jax
import jax.numpy as jnp
from jax import lax
from jax.experimental import pallas as pl
from jax.experimental.pallas import tpu as pltpu
from jax.experimental.pallas import tpu_sc as plsc

VOCAB = 100000
EMBED_DIM = 64
BATCH = 4096
SEQ = 200

NC = 2
NS = 16
NW = NC * NS
TOTAL = BATCH * SEQ
B_PER_W = TOTAL // NW
CHUNK = 128
NCHUNK = B_PER_W // CHUNK
NBUF = 8
NGROUP = NCHUNK // NBUF


def _emb_kernel(idx_hbm, table_hbm, out_hbm, idx_v, rows_v, gsems, osems):
    wid = lax.axis_index("s") * NC + lax.axis_index("c")
    base = wid * B_PER_W

    pltpu.sync_copy(idx_hbm.at[wid], idx_v)

    def fire_gather(c, b):
        return pltpu.async_copy(table_hbm.at[idx_v.at[c]], rows_v.at[b],
                                gsems.at[b])

    def fire_out(c, b):
        return pltpu.async_copy(
            rows_v.at[b], out_hbm.at[pl.ds(base + c * CHUNK, CHUNK)],
            osems.at[b])

    def wait_gather(b):
        pltpu.make_async_copy(table_hbm.at[idx_v.at[0]], rows_v.at[b],
                              gsems.at[b]).wait()

    def wait_out(b):
        pltpu.make_async_copy(rows_v.at[b],
                              out_hbm.at[pl.ds(base, CHUNK)],
                              osems.at[b]).wait()

    for b in range(NBUF):
        fire_gather(b, b)

    def body(g, _):
        for b in range(NBUF):
            wait_gather(b)
            fire_out(g * NBUF + b, b)
        for b in range(NBUF):
            wait_out(b)
            fire_gather((g + 1) * NBUF + b, b)
        return 0

    lax.fori_loop(0, NGROUP - 1, body, 0, unroll=False)

    for b in range(NBUF):
        wait_gather(b)
        fire_out((NGROUP - 1) * NBUF + b, b)
    for b in range(NBUF):
        wait_out(b)


@jax.jit
def kernel(tokenized_sentence, table):
    idx = tokenized_sentence.astype(jnp.int32).reshape(NW, NCHUNK, CHUNK)
    mesh = plsc.VectorSubcoreMesh(core_axis_name="c", subcore_axis_name="s")
    out = pl.kernel(
        _emb_kernel,
        out_type=jax.ShapeDtypeStruct((TOTAL, EMBED_DIM), jnp.float32),
        mesh=mesh,
        scratch_types=[
            pltpu.VMEM((NCHUNK, CHUNK), jnp.int32),
            pltpu.VMEM((NBUF, CHUNK, EMBED_DIM), jnp.float32),
            pltpu.SemaphoreType.DMA((NBUF,)),
            pltpu.SemaphoreType.DMA((NBUF,)),
        ],
        compiler_params=pltpu.CompilerParams(use_tc_tiling_on_sc=False),
    )(idx, table)
    return out.reshape(BATCH, SEQ, EMBED_DIM)


# speedup vs baseline: 4.2653x; 1.0045x over previous
import functools

import jax
import jax.numpy as jnp
from jax import lax
from jax.experimental import pallas as pl
from jax.experimental.pallas import tpu as pltpu
from jax.experimental.pallas import tpu_sc as plsc

VOCAB = 100000
EMBED_DIM = 64
BATCH = 4096
SEQ = 200

NC = 2
NS = 16
NW = NC * NS
TOTAL = BATCH * SEQ
B_PER_W = TOTAL // NW
CHUNK = 128
NCHUNK = B_PER_W // CHUNK
NBUF = 8
NGROUP = NCHUNK // NBUF


GCHUNK = 4
GROUP = GCHUNK * CHUNK
NGROUP2 = B_PER_W // GROUP
NRING = 2


def _emb_kernel(idx_hbm, table_hbm, out_hbm, idx_v, rows_v, gsems, osems):
    wid = lax.axis_index("s") * NC + lax.axis_index("c")
    base = wid * B_PER_W

    pltpu.sync_copy(idx_hbm.at[wid], idx_v)

    def fire_gathers(g, b):
        for q in range(GCHUNK):
            pltpu.async_copy(table_hbm.at[idx_v.at[g * GCHUNK + q]],
                             rows_v.at[b, pl.ds(q * CHUNK, CHUNK)],
                             gsems.at[b])

    def wait_gathers(b):
        for q in range(GCHUNK):
            pltpu.make_async_copy(table_hbm.at[idx_v.at[0]],
                                  rows_v.at[b, pl.ds(0, CHUNK)],
                                  gsems.at[b]).wait()

    def fire_out(g, b):
        pltpu.async_copy(rows_v.at[b],
                         out_hbm.at[pl.ds(base + g * GROUP, GROUP)],
                         osems.at[b])

    def wait_out(b):
        pltpu.make_async_copy(rows_v.at[b],
                              out_hbm.at[pl.ds(base, GROUP)],
                              osems.at[b]).wait()

    for b in range(NRING):
        fire_gathers(b, b)

    def body(p, _):
        for b in range(NRING):
            g = p * NRING + b
            wait_gathers(b)
            fire_out(g, b)
            wait_out(b)
            fire_gathers(g + NRING, b)
        return 0

    lax.fori_loop(0, NGROUP2 // NRING - 1, body, 0, unroll=False)

    for b in range(NRING):
        wait_gathers(b)
        fire_out(NGROUP2 - NRING + b, b)
    for b in range(NRING):
        wait_out(b)


@jax.jit
def kernel(tokenized_sentence, table):
    idx = tokenized_sentence.astype(jnp.int32).reshape(NW, NCHUNK, CHUNK)
    mesh = plsc.VectorSubcoreMesh(core_axis_name="c", subcore_axis_name="s")
    out = pl.kernel(
        _emb_kernel,
        out_type=jax.ShapeDtypeStruct((TOTAL, EMBED_DIM), jnp.float32),
        mesh=mesh,
        scratch_types=[
            pltpu.VMEM((NCHUNK, CHUNK), jnp.int32),
            pltpu.VMEM((NRING, GROUP, EMBED_DIM), jnp.float32),
            pltpu.SemaphoreType.DMA((NRING,)),
            pltpu.SemaphoreType.DMA((NRING,)),
        ],
        compiler_params=pltpu.CompilerParams(use_tc_tiling_on_sc=False),
    )(idx, table)
    return out.reshape(BATCH, SEQ, EMBED_DIM)
